# baseline (speedup 1.0000x reference)
import jax
import jax.numpy as jnp
from jax.experimental import pallas as pl
from jax.experimental.pallas import tpu as pltpu

DIM = 256
NUM_HEADS = 16
N_WIN = 8
TOPK = 4
QK_DIM = DIM
SCALE = QK_DIM ** -0.5
P2 = N_WIN * N_WIN
W2 = 16
CH = QK_DIM // NUM_HEADS
WG = 8


def _stage_a(x_ref, g_ref, b_ref, w16_ref, wqk_ref, qb_ref,
             q_out, kv_out, idx_out):
    x = x_ref[0]
    mu = jnp.mean(x, axis=-1, keepdims=True)
    xc = x - mu
    var = jnp.mean(xc * xc, axis=-1, keepdims=True)
    xn = xc * jax.lax.rsqrt(var + 1e-6) * g_ref[...] + b_ref[...]
    qkv = jnp.dot(xn.astype(jnp.bfloat16), w16_ref[...],
                  preferred_element_type=jnp.float32)
    qkv = qkv + qb_ref[...]
    qkv16 = qkv.astype(jnp.bfloat16).reshape(P2, W2, 3 * DIM)
    q_out[0] = qkv16[..., :QK_DIM]
    kv_out[0] = qkv16[..., QK_DIM:]
    xm = jnp.mean(xn.reshape(P2, W2, DIM), axis=1)
    qk_win = (jnp.dot(xm, wqk_ref[...], preferred_element_type=jnp.float32)
              + qb_ref[:2 * QK_DIM])
    logit = jax.lax.dot_general(
        qk_win[:, :QK_DIM] * SCALE, qk_win[:, QK_DIM:],
        (((1,), (1,)), ((), ())), preferred_element_type=jnp.float32)
    col = jax.lax.broadcasted_iota(jnp.int32, (P2, P2), 1)
    l = logit
    for t in range(TOPK):
        m = jnp.max(l, axis=-1, keepdims=True)
        cand = jnp.where(l == m, col, P2)
        a = jnp.min(cand, axis=-1, keepdims=True)
        idx_out[0, :, t] = a[:, 0]
        l = jnp.where(col == a, -jnp.inf, l)


def _stage_b(idx_ref, q_ref, kv_ref, o_ref, v_ref):
    b = pl.program_id(0)
    g = pl.program_id(1)
    hmask = (jax.lax.broadcasted_iota(jnp.int32, (NUM_HEADS, W2, QK_DIM), 2)
             // CH
             == jax.lax.broadcasted_iota(jnp.int32, (NUM_HEADS, W2, QK_DIM), 0))
    os = []
    for kk in range(WG):
        w = g * WG + kk
        q = q_ref[0, kk]
        ks = []
        vs = []
        for t in range(TOPK):
            s = idx_ref[b, w, t]
            ks.append(kv_ref[0, s, :, :QK_DIM])
            vs.append(kv_ref[0, s, :, QK_DIM:])
        k_sel = jnp.concatenate(ks, axis=0)
        v_sel = jnp.concatenate(vs, axis=0)
        q_stack = jnp.where(
            hmask, jnp.broadcast_to(q[None], (NUM_HEADS, W2, QK_DIM)),
            jnp.bfloat16(0)).reshape(NUM_HEADS * W2, QK_DIM)
        s_all = jax.lax.dot_general(
            q_stack, k_sel,
            (((1,), (1,)), ((), ())),
            preferred_element_type=jnp.float32) * SCALE
        m = jnp.max(s_all, axis=-1, keepdims=True)
        e = jnp.exp(s_all - m)
        p = (e / jnp.sum(e, axis=-1, keepdims=True)).astype(jnp.bfloat16)
        obig = jnp.dot(p, v_sel, preferred_element_type=jnp.float32)
        o = jnp.sum(jnp.where(hmask, obig.reshape(NUM_HEADS, W2, DIM), 0.0),
                    axis=0)
        os.append(o.reshape(4, 4, DIM))
    oall = jnp.stack(os, axis=0)
    o_ref[0, 0] = oall.transpose(1, 0, 2, 3)
    vg = kv_ref[0, pl.ds(g * WG, WG), :, QK_DIM:]
    v_ref[0, 0] = vg.reshape(WG, 4, 4, DIM).transpose(1, 0, 2, 3)


def _stage_c(x_ref, o_ref, v_ref, lw_ref, lb_ref, g1_ref, g2_ref, ln2g_ref,
             ln2b_ref, w1_ref, b1_ref, w2_ref, b2_ref, out_ref):
    H = W = 4 * N_WIN
    v = v_ref[0].astype(jnp.float32).reshape(H, W, DIM)
    x_im = x_ref[0].T
    lw = lw_ref[...]
    row = jax.lax.broadcasted_iota(jnp.int32, (H, W, 1), 0)
    colx = jax.lax.broadcasted_iota(jnp.int32, (H, W, 1), 1)
    vx = []
    for dx in range(5):
        sx = dx - 2
        r = jnp.roll(v, -sx, axis=1) if sx != 0 else v
        valid = jnp.logical_and(colx + sx >= 0, colx + sx < W)
        vx.append(jnp.where(valid, r, 0.0))
    lepe = jnp.zeros((H, W, DIM), jnp.float32)
    for dy in range(5):
        sy = dy - 2
        validy = jnp.logical_and(row + sy >= 0, row + sy < H)
        for dx in range(5):
            r = jnp.roll(vx[dx], -sy, axis=0) if sy != 0 else vx[dx]
            lepe = lepe + jnp.where(validy, r, 0.0) * lw[dy, dx]
    lepe = (lepe + lb_ref[...]).reshape(H * W, DIM)
    xh = x_im + g1_ref[...] * (o_ref[0] + lepe)
    mu = jnp.mean(xh, axis=-1, keepdims=True)
    xc = xh - mu
    var = jnp.mean(xc * xc, axis=-1, keepdims=True)
    y = xc * jax.lax.rsqrt(var + 1e-6) * ln2g_ref[...] + ln2b_ref[...]
    h1 = jnp.dot(y.astype(jnp.bfloat16), w1_ref[...],
                 preferred_element_type=jnp.float32) + b1_ref[...]
    gg = 0.5 * h1 * (1.0 + jax.lax.erf(h1 * (2.0 ** -0.5)))
    y2 = jnp.dot(gg.astype(jnp.bfloat16), w2_ref[...],
                 preferred_element_type=jnp.float32) + b2_ref[...]
    out_ref[0] = (xh + g2_ref[...] * y2).T


def kernel(x, ln1_g, ln1_b, qkv_w, qkv_b, lepe_w, lepe_b, gamma1, gamma2,
           ln2_g, ln2_b, mlp_w1, mlp_b1, mlp_w2, mlp_b2):
    n = x.shape[0]
    H = W = 4 * N_WIN
    x_win = (x.reshape(n, DIM, N_WIN, 4, N_WIN, 4)
             .transpose(0, 2, 4, 3, 5, 1).reshape(n, P2 * W2, DIM))

    q4, kv4, idx = pl.pallas_call(
        _stage_a,
        grid=(n,),
        in_specs=[
            pl.BlockSpec((1, P2 * W2, DIM), lambda b: (b, 0, 0)),
            pl.BlockSpec((DIM,), lambda b: (0,)),
            pl.BlockSpec((DIM,), lambda b: (0,)),
            pl.BlockSpec((DIM, 3 * DIM), lambda b: (0, 0)),
            pl.BlockSpec((DIM, 2 * QK_DIM), lambda b: (0, 0)),
            pl.BlockSpec((3 * DIM,), lambda b: (0,)),
        ],
        out_specs=[
            pl.BlockSpec((1, P2, W2, QK_DIM), lambda b: (b, 0, 0, 0)),
            pl.BlockSpec((1, P2, W2, 2 * DIM), lambda b: (b, 0, 0, 0)),
            pl.BlockSpec((1, P2, TOPK), lambda b: (b, 0, 0)),
        ],
        out_shape=[
            jax.ShapeDtypeStruct((n, P2, W2, QK_DIM), jnp.bfloat16),
            jax.ShapeDtypeStruct((n, P2, W2, 2 * DIM), jnp.bfloat16),
            jax.ShapeDtypeStruct((n, P2, TOPK), jnp.int32),
        ],
    )(x_win, ln1_g, ln1_b, qkv_w.astype(jnp.bfloat16),
      qkv_w[:, :2 * QK_DIM], qkv_b)

    o6, v6 = pl.pallas_call(
        _stage_b,
        grid=(n, N_WIN),
        in_specs=[
            pl.BlockSpec(memory_space=pltpu.SMEM),
            pl.BlockSpec((1, WG, W2, QK_DIM), lambda b, g: (b, g, 0, 0)),
            pl.BlockSpec((1, P2, W2, 2 * DIM), lambda b, g: (b, 0, 0, 0)),
        ],
        out_specs=[
            pl.BlockSpec((1, 1, 4, N_WIN, 4, DIM),
                         lambda b, g: (b, g, 0, 0, 0, 0)),
            pl.BlockSpec((1, 1, 4, N_WIN, 4, DIM),
                         lambda b, g: (b, g, 0, 0, 0, 0)),
        ],
        out_shape=[
            jax.ShapeDtypeStruct((n, N_WIN, 4, N_WIN, 4, DIM), jnp.float32),
            jax.ShapeDtypeStruct((n, N_WIN, 4, N_WIN, 4, DIM), jnp.bfloat16),
        ],
    )(idx, q4, kv4)

    o_img = o6.reshape(n, H * W, DIM)
    v_img = v6.reshape(n, H * W, DIM)
    x_flat = x.reshape(n, DIM, H * W)
    lw = lepe_w.reshape(DIM, 5, 5).transpose(1, 2, 0)

    out = pl.pallas_call(
        _stage_c,
        grid=(n,),
        in_specs=[
            pl.BlockSpec((1, DIM, H * W), lambda b: (b, 0, 0)),
            pl.BlockSpec((1, H * W, DIM), lambda b: (b, 0, 0)),
            pl.BlockSpec((1, H * W, DIM), lambda b: (b, 0, 0)),
            pl.BlockSpec((5, 5, DIM), lambda b: (0, 0, 0)),
            pl.BlockSpec((DIM,), lambda b: (0,)),
            pl.BlockSpec((DIM,), lambda b: (0,)),
            pl.BlockSpec((DIM,), lambda b: (0,)),
            pl.BlockSpec((DIM,), lambda b: (0,)),
            pl.BlockSpec((DIM,), lambda b: (0,)),
            pl.BlockSpec((DIM, 4 * DIM), lambda b: (0, 0)),
            pl.BlockSpec((4 * DIM,), lambda b: (0,)),
            pl.BlockSpec((4 * DIM, DIM), lambda b: (0, 0)),
            pl.BlockSpec((DIM,), lambda b: (0,)),
        ],
        out_specs=pl.BlockSpec((1, DIM, H * W), lambda b: (b, 0, 0)),
        out_shape=jax.ShapeDtypeStruct((n, DIM, H * W), jnp.float32),
    )(x_flat, o_img, v_img, lw, lepe_b, gamma1, gamma2, ln2_g, ln2_b,
      mlp_w1.astype(jnp.bfloat16), mlp_b1, mlp_w2.astype(jnp.bfloat16),
      mlp_b2)

    return out.reshape(n, DIM, H, W)


# speedup vs baseline: 1.2891x; 1.2891x over previous
import jax
import jax.numpy as jnp
from jax.experimental import pallas as pl
from jax.experimental.pallas import tpu as pltpu

DIM = 256
NUM_HEADS = 16
N_WIN = 8
TOPK = 4
QK_DIM = DIM
SCALE = QK_DIM ** -0.5
P2 = N_WIN * N_WIN
W2 = 16
CH = QK_DIM // NUM_HEADS
WG = 16


def _stage_a(x_ref, g_ref, b_ref, w16_ref, wqk_ref, qb_ref,
             q_out, kv_out, idx_out):
    x = x_ref[0].T
    mu = jnp.mean(x, axis=-1, keepdims=True)
    xc = x - mu
    var = jnp.mean(xc * xc, axis=-1, keepdims=True)
    xn = xc * jax.lax.rsqrt(var + 1e-6) * g_ref[...] + b_ref[...]
    qkv = jnp.dot(xn.astype(jnp.bfloat16), w16_ref[...],
                  preferred_element_type=jnp.float32)
    qkv = qkv + qb_ref[...]
    qkv16 = (qkv.astype(jnp.bfloat16)
             .reshape(N_WIN, 4, N_WIN, 4, 3 * DIM)
             .transpose(0, 2, 1, 3, 4).reshape(P2, W2, 3 * DIM))
    q_out[0] = qkv16[..., :QK_DIM]
    kv_out[0] = qkv16[..., QK_DIM:]
    xm = jnp.mean(xn.reshape(N_WIN, 4, N_WIN, 4, DIM),
                  axis=(1, 3)).reshape(P2, DIM)
    qk_win = (jnp.dot(xm, wqk_ref[...], preferred_element_type=jnp.float32)
              + qb_ref[:2 * QK_DIM])
    logit = jax.lax.dot_general(
        qk_win[:, :QK_DIM] * SCALE, qk_win[:, QK_DIM:],
        (((1,), (1,)), ((), ())), preferred_element_type=jnp.float32)
    col = jax.lax.broadcasted_iota(jnp.int32, (P2, P2), 1)
    l = logit
    for t in range(TOPK):
        m = jnp.max(l, axis=-1, keepdims=True)
        cand = jnp.where(l == m, col, P2)
        a = jnp.min(cand, axis=-1, keepdims=True)
        idx_out[0, :, t] = a[:, 0]
        l = jnp.where(col == a, -jnp.inf, l)


def _stage_b(idx_ref, q_ref, kv_ref, o_ref, v_ref):
    b = pl.program_id(0)
    g = pl.program_id(1)
    hmask = (jax.lax.broadcasted_iota(jnp.int32, (NUM_HEADS, W2, QK_DIM), 2)
             // CH
             == jax.lax.broadcasted_iota(jnp.int32, (NUM_HEADS, W2, QK_DIM), 0))
    os = []
    for kk in range(WG):
        w = g * WG + kk
        q = q_ref[0, kk]
        ks = []
        vs = []
        for t in range(TOPK):
            s = idx_ref[b, w, t]
            ks.append(kv_ref[0, s, :, :QK_DIM])
            vs.append(kv_ref[0, s, :, QK_DIM:])
        k_sel = jnp.concatenate(ks, axis=0)
        v_sel = jnp.concatenate(vs, axis=0)
        q_stack = jnp.where(
            hmask, jnp.broadcast_to(q[None], (NUM_HEADS, W2, QK_DIM)),
            jnp.bfloat16(0)).reshape(NUM_HEADS * W2, QK_DIM)
        s_all = jax.lax.dot_general(
            q_stack, k_sel,
            (((1,), (1,)), ((), ())),
            preferred_element_type=jnp.float32) * SCALE
        m = jnp.max(s_all, axis=-1, keepdims=True)
        e = jnp.exp(s_all - m)
        p = (e / jnp.sum(e, axis=-1, keepdims=True)).astype(jnp.bfloat16)
        obig = jnp.dot(p, v_sel, preferred_element_type=jnp.float32)
        o = jnp.sum(jnp.where(hmask, obig.reshape(NUM_HEADS, W2, DIM), 0.0),
                    axis=0)
        os.append(o.reshape(4, 4, DIM))
    oall = jnp.stack(os, axis=0)
    o_ref[0] = (oall.reshape(2, N_WIN, 4, 4, DIM)
                .transpose(0, 2, 1, 3, 4))
    vg = kv_ref[0, pl.ds(g * WG, WG), :, QK_DIM:]
    v_ref[0] = (vg.reshape(2, N_WIN, 4, 4, DIM)
                .transpose(0, 2, 1, 3, 4))


def _stage_c(x_ref, o_ref, v_ref, lw_ref, lb_ref, g1_ref, g2_ref, ln2g_ref,
             ln2b_ref, w1_ref, b1_ref, w2_ref, b2_ref, out_ref):
    H = W = 4 * N_WIN
    v = v_ref[0].astype(jnp.float32).reshape(H, W, DIM)
    x_im = x_ref[0].T
    lw = lw_ref[...]
    row = jax.lax.broadcasted_iota(jnp.int32, (H, W, 1), 0)
    colx = jax.lax.broadcasted_iota(jnp.int32, (H, W, 1), 1)
    vx = []
    for dx in range(5):
        sx = dx - 2
        r = jnp.roll(v, -sx, axis=1) if sx != 0 else v
        valid = jnp.logical_and(colx + sx >= 0, colx + sx < W)
        vx.append(jnp.where(valid, r, 0.0))
    lepe = jnp.zeros((H, W, DIM), jnp.float32)
    for dy in range(5):
        sy = dy - 2
        validy = jnp.logical_and(row + sy >= 0, row + sy < H)
        for dx in range(5):
            r = jnp.roll(vx[dx], -sy, axis=0) if sy != 0 else vx[dx]
            lepe = lepe + jnp.where(validy, r, 0.0) * lw[dy, dx]
    lepe = (lepe + lb_ref[...]).reshape(H * W, DIM)
    xh = x_im + g1_ref[...] * (o_ref[0] + lepe)
    mu = jnp.mean(xh, axis=-1, keepdims=True)
    xc = xh - mu
    var = jnp.mean(xc * xc, axis=-1, keepdims=True)
    y = xc * jax.lax.rsqrt(var + 1e-6) * ln2g_ref[...] + ln2b_ref[...]
    h1 = jnp.dot(y.astype(jnp.bfloat16), w1_ref[...],
                 preferred_element_type=jnp.float32) + b1_ref[...]
    gg = 0.5 * h1 * (1.0 + jax.lax.erf(h1 * (2.0 ** -0.5)))
    y2 = jnp.dot(gg.astype(jnp.bfloat16), w2_ref[...],
                 preferred_element_type=jnp.float32) + b2_ref[...]
    out_ref[0] = (xh + g2_ref[...] * y2).T


def kernel(x, ln1_g, ln1_b, qkv_w, qkv_b, lepe_w, lepe_b, gamma1, gamma2,
           ln2_g, ln2_b, mlp_w1, mlp_b1, mlp_w2, mlp_b2):
    n = x.shape[0]
    H = W = 4 * N_WIN
    x_flat = x.reshape(n, DIM, H * W)

    q4, kv4, idx = pl.pallas_call(
        _stage_a,
        grid=(n,),
        in_specs=[
            pl.BlockSpec((1, DIM, H * W), lambda b: (b, 0, 0)),
            pl.BlockSpec((DIM,), lambda b: (0,)),
            pl.BlockSpec((DIM,), lambda b: (0,)),
            pl.BlockSpec((DIM, 3 * DIM), lambda b: (0, 0)),
            pl.BlockSpec((DIM, 2 * QK_DIM), lambda b: (0, 0)),
            pl.BlockSpec((3 * DIM,), lambda b: (0,)),
        ],
        out_specs=[
            pl.BlockSpec((1, P2, W2, QK_DIM), lambda b: (b, 0, 0, 0)),
            pl.BlockSpec((1, P2, W2, 2 * DIM), lambda b: (b, 0, 0, 0)),
            pl.BlockSpec((1, P2, TOPK), lambda b: (b, 0, 0)),
        ],
        out_shape=[
            jax.ShapeDtypeStruct((n, P2, W2, QK_DIM), jnp.bfloat16),
            jax.ShapeDtypeStruct((n, P2, W2, 2 * DIM), jnp.bfloat16),
            jax.ShapeDtypeStruct((n, P2, TOPK), jnp.int32),
        ],
    )(x_flat, ln1_g, ln1_b, qkv_w.astype(jnp.bfloat16),
      qkv_w[:, :2 * QK_DIM], qkv_b)

    o6, v6 = pl.pallas_call(
        _stage_b,
        grid=(n, P2 // WG),
        in_specs=[
            pl.BlockSpec(memory_space=pltpu.SMEM),
            pl.BlockSpec((1, WG, W2, QK_DIM), lambda b, g: (b, g, 0, 0)),
            pl.BlockSpec((1, P2, W2, 2 * DIM), lambda b, g: (b, 0, 0, 0)),
        ],
        out_specs=[
            pl.BlockSpec((1, 2, 4, N_WIN, 4, DIM),
                         lambda b, g: (b, g, 0, 0, 0, 0)),
            pl.BlockSpec((1, 2, 4, N_WIN, 4, DIM),
                         lambda b, g: (b, g, 0, 0, 0, 0)),
        ],
        out_shape=[
            jax.ShapeDtypeStruct((n, N_WIN, 4, N_WIN, 4, DIM), jnp.float32),
            jax.ShapeDtypeStruct((n, N_WIN, 4, N_WIN, 4, DIM), jnp.bfloat16),
        ],
    )(idx, q4, kv4)

    o_img = o6.reshape(n, H * W, DIM)
    v_img = v6.reshape(n, H * W, DIM)
    lw = lepe_w.reshape(DIM, 5, 5).transpose(1, 2, 0)

    out = pl.pallas_call(
        _stage_c,
        grid=(n,),
        in_specs=[
            pl.BlockSpec((1, DIM, H * W), lambda b: (b, 0, 0)),
            pl.BlockSpec((1, H * W, DIM), lambda b: (b, 0, 0)),
            pl.BlockSpec((1, H * W, DIM), lambda b: (b, 0, 0)),
            pl.BlockSpec((5, 5, DIM), lambda b: (0, 0, 0)),
            pl.BlockSpec((DIM,), lambda b: (0,)),
            pl.BlockSpec((DIM,), lambda b: (0,)),
            pl.BlockSpec((DIM,), lambda b: (0,)),
            pl.BlockSpec((DIM,), lambda b: (0,)),
            pl.BlockSpec((DIM,), lambda b: (0,)),
            pl.BlockSpec((DIM, 4 * DIM), lambda b: (0, 0)),
            pl.BlockSpec((4 * DIM,), lambda b: (0,)),
            pl.BlockSpec((4 * DIM, DIM), lambda b: (0, 0)),
            pl.BlockSpec((DIM,), lambda b: (0,)),
        ],
        out_specs=pl.BlockSpec((1, DIM, H * W), lambda b: (b, 0, 0)),
        out_shape=jax.ShapeDtypeStruct((n, DIM, H * W), jnp.float32),
    )(x_flat, o_img, v_img, lw, lepe_b, gamma1, gamma2, ln2_g, ln2_b,
      mlp_w1.astype(jnp.bfloat16), mlp_b1, mlp_w2.astype(jnp.bfloat16),
      mlp_b2)

    return out.reshape(n, DIM, H, W)


# speedup vs baseline: 1.4291x; 1.1086x over previous
import jax
import jax.numpy as jnp
from jax.experimental import pallas as pl
from jax.experimental.pallas import tpu as pltpu

DIM = 256
NUM_HEADS = 16
N_WIN = 8
TOPK = 4
QK_DIM = DIM
SCALE = QK_DIM ** -0.5
P2 = N_WIN * N_WIN
W2 = 16
CH = QK_DIM // NUM_HEADS
WG = 32
WROW = WG // N_WIN


def _stage_a(x_ref, g_ref, b_ref, w16_ref, wqk_ref, qb_ref,
             q_out, kv_out, idx_out):
    x = x_ref[0].T
    mu = jnp.mean(x, axis=-1, keepdims=True)
    xc = x - mu
    var = jnp.mean(xc * xc, axis=-1, keepdims=True)
    xn = xc * jax.lax.rsqrt(var + 1e-6) * g_ref[...] + b_ref[...]
    qkv = jnp.dot(xn.astype(jnp.bfloat16), w16_ref[...],
                  preferred_element_type=jnp.float32)
    qkv = qkv + qb_ref[...]
    qkv16 = (qkv.astype(jnp.bfloat16)
             .reshape(N_WIN, 4, N_WIN, 4, 3 * DIM)
             .transpose(0, 2, 1, 3, 4).reshape(P2, W2, 3 * DIM))
    q_out[0] = qkv16[..., :QK_DIM]
    kv_out[0] = qkv16[..., QK_DIM:]
    xm = jnp.mean(xn.reshape(N_WIN, 4, N_WIN, 4, DIM),
                  axis=(1, 3)).reshape(P2, DIM)
    qk_win = (jnp.dot(xm, wqk_ref[...], preferred_element_type=jnp.float32)
              + qb_ref[:2 * QK_DIM])
    logit = jax.lax.dot_general(
        qk_win[:, :QK_DIM] * SCALE, qk_win[:, QK_DIM:],
        (((1,), (1,)), ((), ())), preferred_element_type=jnp.float32)
    col = jax.lax.broadcasted_iota(jnp.int32, (P2, P2), 1)
    l = logit
    for t in range(TOPK):
        m = jnp.max(l, axis=-1, keepdims=True)
        cand = jnp.where(l == m, col, P2)
        a = jnp.min(cand, axis=-1, keepdims=True)
        idx_out[0, :, t] = a[:, 0]
        l = jnp.where(col == a, -jnp.inf, l)


def _stage_b(idx_ref, q_ref, kv_ref, o_ref, v_ref):
    b = pl.program_id(0)
    g = pl.program_id(1)
    hmask = (jax.lax.broadcasted_iota(jnp.int32, (NUM_HEADS, W2, QK_DIM), 2)
             // CH
             == jax.lax.broadcasted_iota(jnp.int32, (NUM_HEADS, W2, QK_DIM), 0))
    os = []
    for kk in range(WG):
        w = g * WG + kk
        q = q_ref[0, kk]
        ks = []
        vs = []
        for t in range(TOPK):
            s = idx_ref[b, w, t]
            ks.append(kv_ref[0, s, :, :QK_DIM])
            vs.append(kv_ref[0, s, :, QK_DIM:])
        k_sel = jnp.concatenate(ks, axis=0)
        v_sel = jnp.concatenate(vs, axis=0)
        q_stack = jnp.where(
            hmask, jnp.broadcast_to(q[None], (NUM_HEADS, W2, QK_DIM)),
            jnp.bfloat16(0)).reshape(NUM_HEADS * W2, QK_DIM)
        s_all = jax.lax.dot_general(
            q_stack, k_sel,
            (((1,), (1,)), ((), ())),
            preferred_element_type=jnp.float32) * SCALE
        m = jnp.max(s_all, axis=-1, keepdims=True)
        e = jnp.exp(s_all - m)
        p = (e / jnp.sum(e, axis=-1, keepdims=True)).astype(jnp.bfloat16)
        obig = jnp.dot(p, v_sel, preferred_element_type=jnp.float32)
        o = jnp.sum(jnp.where(hmask, obig.reshape(NUM_HEADS, W2, DIM), 0.0),
                    axis=0)
        os.append(o.reshape(4, 4, DIM))
    oall = jnp.stack(os, axis=0)
    o_ref[0] = (oall.reshape(WROW, N_WIN, 4, 4, DIM)
                .transpose(0, 2, 1, 3, 4)
                .astype(jnp.bfloat16))
    vg = kv_ref[0, pl.ds(g * WG, WG), :, QK_DIM:]
    v_ref[0] = (vg.reshape(WROW, N_WIN, 4, 4, DIM)
                .transpose(0, 2, 1, 3, 4))


def _stage_c(x_ref, o_ref, v_ref, lw_ref, lb_ref, g1_ref, g2_ref, ln2g_ref,
             ln2b_ref, w1_ref, b1_ref, w2_ref, b2_ref, out_ref):
    H = W = 4 * N_WIN
    v = v_ref[0].reshape(H, W, DIM)
    x_im = x_ref[0].T
    lw = lw_ref[...]
    row = jax.lax.broadcasted_iota(jnp.int32, (H, W, 1), 0)
    colx = jax.lax.broadcasted_iota(jnp.int32, (H, W, 1), 1)
    z16 = jnp.bfloat16(0)
    vx = []
    for dx in range(5):
        sx = dx - 2
        r = jnp.roll(v, -sx, axis=1) if sx != 0 else v
        valid = jnp.logical_and(colx + sx >= 0, colx + sx < W)
        vx.append(jnp.where(valid, r, z16))
    lepe = jnp.zeros((H, W, DIM), jnp.bfloat16)
    for dy in range(5):
        sy = dy - 2
        validy = jnp.logical_and(row + sy >= 0, row + sy < H)
        for dx in range(5):
            r = jnp.roll(vx[dx], -sy, axis=0) if sy != 0 else vx[dx]
            lepe = lepe + jnp.where(validy, r, z16) * lw[dy, dx]
    lepe = lepe.astype(jnp.float32).reshape(H * W, DIM) + lb_ref[...]
    xh = x_im + g1_ref[...] * (o_ref[0].astype(jnp.float32) + lepe)
    mu = jnp.mean(xh, axis=-1, keepdims=True)
    xc = xh - mu
    var = jnp.mean(xc * xc, axis=-1, keepdims=True)
    y = xc * jax.lax.rsqrt(var + 1e-6) * ln2g_ref[...] + ln2b_ref[...]
    h1 = jnp.dot(y.astype(jnp.bfloat16), w1_ref[...],
                 preferred_element_type=jnp.float32) + b1_ref[...]
    gg = 0.5 * h1 * (1.0 + jax.lax.erf(h1 * (2.0 ** -0.5)))
    y2 = jnp.dot(gg.astype(jnp.bfloat16), w2_ref[...],
                 preferred_element_type=jnp.float32) + b2_ref[...]
    out_ref[0] = (xh + g2_ref[...] * y2).T


def kernel(x, ln1_g, ln1_b, qkv_w, qkv_b, lepe_w, lepe_b, gamma1, gamma2,
           ln2_g, ln2_b, mlp_w1, mlp_b1, mlp_w2, mlp_b2):
    n = x.shape[0]
    H = W = 4 * N_WIN
    x_flat = x.reshape(n, DIM, H * W)

    q4, kv4, idx = pl.pallas_call(
        _stage_a,
        grid=(n,),
        in_specs=[
            pl.BlockSpec((1, DIM, H * W), lambda b: (b, 0, 0)),
            pl.BlockSpec((DIM,), lambda b: (0,)),
            pl.BlockSpec((DIM,), lambda b: (0,)),
            pl.BlockSpec((DIM, 3 * DIM), lambda b: (0, 0)),
            pl.BlockSpec((DIM, 2 * QK_DIM), lambda b: (0, 0)),
            pl.BlockSpec((3 * DIM,), lambda b: (0,)),
        ],
        out_specs=[
            pl.BlockSpec((1, P2, W2, QK_DIM), lambda b: (b, 0, 0, 0)),
            pl.BlockSpec((1, P2, W2, 2 * DIM), lambda b: (b, 0, 0, 0)),
            pl.BlockSpec((1, P2, TOPK), lambda b: (b, 0, 0)),
        ],
        out_shape=[
            jax.ShapeDtypeStruct((n, P2, W2, QK_DIM), jnp.bfloat16),
            jax.ShapeDtypeStruct((n, P2, W2, 2 * DIM), jnp.bfloat16),
            jax.ShapeDtypeStruct((n, P2, TOPK), jnp.int32),
        ],
    )(x_flat, ln1_g, ln1_b, qkv_w.astype(jnp.bfloat16),
      qkv_w[:, :2 * QK_DIM], qkv_b)

    o6, v6 = pl.pallas_call(
        _stage_b,
        grid=(n, P2 // WG),
        in_specs=[
            pl.BlockSpec(memory_space=pltpu.SMEM),
            pl.BlockSpec((1, WG, W2, QK_DIM), lambda b, g: (b, g, 0, 0)),
            pl.BlockSpec((1, P2, W2, 2 * DIM), lambda b, g: (b, 0, 0, 0)),
        ],
        out_specs=[
            pl.BlockSpec((1, WROW, 4, N_WIN, 4, DIM),
                         lambda b, g: (b, g, 0, 0, 0, 0)),
            pl.BlockSpec((1, WROW, 4, N_WIN, 4, DIM),
                         lambda b, g: (b, g, 0, 0, 0, 0)),
        ],
        out_shape=[
            jax.ShapeDtypeStruct((n, N_WIN, 4, N_WIN, 4, DIM), jnp.bfloat16),
            jax.ShapeDtypeStruct((n, N_WIN, 4, N_WIN, 4, DIM), jnp.bfloat16),
        ],
    )(idx, q4, kv4)

    o_img = o6.reshape(n, H * W, DIM)
    v_img = v6.reshape(n, H * W, DIM)
    lw = (lepe_w.reshape(DIM, 5, 5).transpose(1, 2, 0)
          .astype(jnp.bfloat16))

    out = pl.pallas_call(
        _stage_c,
        grid=(n,),
        in_specs=[
            pl.BlockSpec((1, DIM, H * W), lambda b: (b, 0, 0)),
            pl.BlockSpec((1, H * W, DIM), lambda b: (b, 0, 0)),
            pl.BlockSpec((1, H * W, DIM), lambda b: (b, 0, 0)),
            pl.BlockSpec((5, 5, DIM), lambda b: (0, 0, 0)),
            pl.BlockSpec((DIM,), lambda b: (0,)),
            pl.BlockSpec((DIM,), lambda b: (0,)),
            pl.BlockSpec((DIM,), lambda b: (0,)),
            pl.BlockSpec((DIM,), lambda b: (0,)),
            pl.BlockSpec((DIM,), lambda b: (0,)),
            pl.BlockSpec((DIM, 4 * DIM), lambda b: (0, 0)),
            pl.BlockSpec((4 * DIM,), lambda b: (0,)),
            pl.BlockSpec((4 * DIM, DIM), lambda b: (0, 0)),
            pl.BlockSpec((DIM,), lambda b: (0,)),
        ],
        out_specs=pl.BlockSpec((1, DIM, H * W), lambda b: (b, 0, 0)),
        out_shape=jax.ShapeDtypeStruct((n, DIM, H * W), jnp.float32),
    )(x_flat, o_img, v_img, lw, lepe_b, gamma1, gamma2, ln2_g, ln2_b,
      mlp_w1.astype(jnp.bfloat16), mlp_b1, mlp_w2.astype(jnp.bfloat16),
      mlp_b2)

    return out.reshape(n, DIM, H, W)


# speedup vs baseline: 1.4676x; 1.0270x over previous
import jax
import jax.numpy as jnp
from jax.experimental import pallas as pl
from jax.experimental.pallas import tpu as pltpu

DIM = 256
NUM_HEADS = 16
N_WIN = 8
TOPK = 4
QK_DIM = DIM
SCALE = QK_DIM ** -0.5
P2 = N_WIN * N_WIN
W2 = 16
CH = QK_DIM // NUM_HEADS
WG = 32
WROW = WG // N_WIN


def _stage_a(x_ref, g_ref, b_ref, wf_ref, qb_ref,
             q_out, kv_out, idx_out):
    x = x_ref[0].T
    mu = jnp.mean(x, axis=-1, keepdims=True)
    xc = x - mu
    var = jnp.mean(xc * xc, axis=-1, keepdims=True)
    xn = xc * jax.lax.rsqrt(var + 1e-6) * g_ref[...] + b_ref[...]
    wf = wf_ref[...]
    qkv = jnp.dot(xn.astype(jnp.bfloat16), wf.astype(jnp.bfloat16),
                  preferred_element_type=jnp.float32)
    qkv = qkv + qb_ref[...]
    qkv16 = (qkv.astype(jnp.bfloat16)
             .reshape(N_WIN, 4, N_WIN, 4, 3 * DIM)
             .transpose(0, 2, 1, 3, 4).reshape(P2, W2, 3 * DIM))
    q_out[0] = qkv16[..., :QK_DIM]
    kv_out[0] = qkv16[..., QK_DIM:]
    xm = jnp.mean(xn.reshape(N_WIN, 4, N_WIN, 4, DIM),
                  axis=(1, 3)).reshape(P2, DIM)
    qk_win = (jnp.dot(xm, wf[:, :2 * QK_DIM],
                      preferred_element_type=jnp.float32)
              + qb_ref[:2 * QK_DIM])
    logit = jax.lax.dot_general(
        qk_win[:, :QK_DIM] * SCALE, qk_win[:, QK_DIM:],
        (((1,), (1,)), ((), ())), preferred_element_type=jnp.float32)
    col = jax.lax.broadcasted_iota(jnp.int32, (P2, P2), 1)
    l = logit
    for t in range(TOPK):
        m = jnp.max(l, axis=-1, keepdims=True)
        cand = jnp.where(l == m, col, P2)
        a = jnp.min(cand, axis=-1, keepdims=True)
        idx_out[0, :, t] = a[:, 0]
        l = jnp.where(col == a, -jnp.inf, l)


def _stage_b(idx_ref, q_ref, kv_ref, o_ref, v_ref):
    b = pl.program_id(0)
    g = pl.program_id(1)
    hmask = (jax.lax.broadcasted_iota(jnp.int32, (NUM_HEADS, W2, QK_DIM), 2)
             // CH
             == jax.lax.broadcasted_iota(jnp.int32, (NUM_HEADS, W2, QK_DIM), 0))
    os = []
    for kk in range(WG):
        w = g * WG + kk
        q = q_ref[0, kk]
        ks = []
        vs = []
        for t in range(TOPK):
            s = idx_ref[b, w, t]
            ks.append(kv_ref[0, s, :, :QK_DIM])
            vs.append(kv_ref[0, s, :, QK_DIM:])
        k_sel = jnp.concatenate(ks, axis=0)
        v_sel = jnp.concatenate(vs, axis=0)
        q_stack = jnp.where(
            hmask, jnp.broadcast_to(q[None], (NUM_HEADS, W2, QK_DIM)),
            jnp.bfloat16(0)).reshape(NUM_HEADS * W2, QK_DIM)
        s_all = jax.lax.dot_general(
            q_stack, k_sel,
            (((1,), (1,)), ((), ())),
            preferred_element_type=jnp.float32) * SCALE
        m = jnp.max(s_all, axis=-1, keepdims=True)
        e = jnp.exp(s_all - m)
        p = (e / jnp.sum(e, axis=-1, keepdims=True)).astype(jnp.bfloat16)
        obig = jnp.dot(p, v_sel, preferred_element_type=jnp.float32)
        o = jnp.sum(jnp.where(hmask, obig.reshape(NUM_HEADS, W2, DIM), 0.0),
                    axis=0)
        os.append(o.reshape(4, 4, DIM))
    oall = jnp.stack(os, axis=0)
    o_ref[0] = (oall.reshape(WROW, N_WIN, 4, 4, DIM)
                .transpose(0, 2, 1, 3, 4)
                .astype(jnp.bfloat16))
    vg = kv_ref[0, pl.ds(g * WG, WG), :, QK_DIM:]
    v_ref[0] = (vg.reshape(WROW, N_WIN, 4, 4, DIM)
                .transpose(0, 2, 1, 3, 4))


def _stage_c(x_ref, o_ref, v_ref, lw_ref, lb_ref, g1_ref, g2_ref, ln2g_ref,
             ln2b_ref, w1_ref, b1_ref, w2_ref, b2_ref, out_ref):
    H = W = 4 * N_WIN
    v = v_ref[0].reshape(H, W, DIM)
    x_im = x_ref[0].T
    lw = lw_ref[...]
    row = jax.lax.broadcasted_iota(jnp.int32, (H, W, 1), 0)
    colx = jax.lax.broadcasted_iota(jnp.int32, (H, W, 1), 1)
    z16 = jnp.bfloat16(0)
    vx = []
    for dx in range(5):
        sx = dx - 2
        r = jnp.roll(v, -sx, axis=1) if sx != 0 else v
        valid = jnp.logical_and(colx + sx >= 0, colx + sx < W)
        vx.append(jnp.where(valid, r, z16))
    lepe = jnp.zeros((H, W, DIM), jnp.bfloat16)
    for dy in range(5):
        sy = dy - 2
        validy = jnp.logical_and(row + sy >= 0, row + sy < H)
        for dx in range(5):
            r = jnp.roll(vx[dx], -sy, axis=0) if sy != 0 else vx[dx]
            lepe = lepe + jnp.where(validy, r, z16) * lw[dy, dx]
    lepe = lepe.astype(jnp.float32).reshape(H * W, DIM) + lb_ref[...]
    xh = x_im + g1_ref[...] * (o_ref[0].astype(jnp.float32) + lepe)
    mu = jnp.mean(xh, axis=-1, keepdims=True)
    xc = xh - mu
    var = jnp.mean(xc * xc, axis=-1, keepdims=True)
    y = xc * jax.lax.rsqrt(var + 1e-6) * ln2g_ref[...] + ln2b_ref[...]
    h1 = jnp.dot(y.astype(jnp.bfloat16), w1_ref[...].astype(jnp.bfloat16),
                 preferred_element_type=jnp.float32) + b1_ref[...]
    gg = 0.5 * h1 * (1.0 + jax.lax.erf(h1 * (2.0 ** -0.5)))
    y2 = jnp.dot(gg.astype(jnp.bfloat16), w2_ref[...].astype(jnp.bfloat16),
                 preferred_element_type=jnp.float32) + b2_ref[...]
    out_ref[0] = (xh + g2_ref[...] * y2).T


def kernel(x, ln1_g, ln1_b, qkv_w, qkv_b, lepe_w, lepe_b, gamma1, gamma2,
           ln2_g, ln2_b, mlp_w1, mlp_b1, mlp_w2, mlp_b2):
    n = x.shape[0]
    H = W = 4 * N_WIN
    x_flat = x.reshape(n, DIM, H * W)

    q4, kv4, idx = pl.pallas_call(
        _stage_a,
        grid=(n,),
        in_specs=[
            pl.BlockSpec((1, DIM, H * W), lambda b: (b, 0, 0)),
            pl.BlockSpec((DIM,), lambda b: (0,)),
            pl.BlockSpec((DIM,), lambda b: (0,)),
            pl.BlockSpec((DIM, 3 * DIM), lambda b: (0, 0)),
            pl.BlockSpec((3 * DIM,), lambda b: (0,)),
        ],
        out_specs=[
            pl.BlockSpec((1, P2, W2, QK_DIM), lambda b: (b, 0, 0, 0)),
            pl.BlockSpec((1, P2, W2, 2 * DIM), lambda b: (b, 0, 0, 0)),
            pl.BlockSpec((1, P2, TOPK), lambda b: (b, 0, 0)),
        ],
        out_shape=[
            jax.ShapeDtypeStruct((n, P2, W2, QK_DIM), jnp.bfloat16),
            jax.ShapeDtypeStruct((n, P2, W2, 2 * DIM), jnp.bfloat16),
            jax.ShapeDtypeStruct((n, P2, TOPK), jnp.int32),
        ],
    )(x_flat, ln1_g, ln1_b, qkv_w, qkv_b)

    o6, v6 = pl.pallas_call(
        _stage_b,
        grid=(n, P2 // WG),
        in_specs=[
            pl.BlockSpec(memory_space=pltpu.SMEM),
            pl.BlockSpec((1, WG, W2, QK_DIM), lambda b, g: (b, g, 0, 0)),
            pl.BlockSpec((1, P2, W2, 2 * DIM), lambda b, g: (b, 0, 0, 0)),
        ],
        out_specs=[
            pl.BlockSpec((1, WROW, 4, N_WIN, 4, DIM),
                         lambda b, g: (b, g, 0, 0, 0, 0)),
            pl.BlockSpec((1, WROW, 4, N_WIN, 4, DIM),
                         lambda b, g: (b, g, 0, 0, 0, 0)),
        ],
        out_shape=[
            jax.ShapeDtypeStruct((n, N_WIN, 4, N_WIN, 4, DIM), jnp.bfloat16),
            jax.ShapeDtypeStruct((n, N_WIN, 4, N_WIN, 4, DIM), jnp.bfloat16),
        ],
    )(idx, q4, kv4)

    o_img = o6.reshape(n, H * W, DIM)
    v_img = v6.reshape(n, H * W, DIM)
    lw = (lepe_w.reshape(DIM, 5, 5).transpose(1, 2, 0)
          .astype(jnp.bfloat16))

    out = pl.pallas_call(
        _stage_c,
        grid=(n,),
        in_specs=[
            pl.BlockSpec((1, DIM, H * W), lambda b: (b, 0, 0)),
            pl.BlockSpec((1, H * W, DIM), lambda b: (b, 0, 0)),
            pl.BlockSpec((1, H * W, DIM), lambda b: (b, 0, 0)),
            pl.BlockSpec((5, 5, DIM), lambda b: (0, 0, 0)),
            pl.BlockSpec((DIM,), lambda b: (0,)),
            pl.BlockSpec((DIM,), lambda b: (0,)),
            pl.BlockSpec((DIM,), lambda b: (0,)),
            pl.BlockSpec((DIM,), lambda b: (0,)),
            pl.BlockSpec((DIM,), lambda b: (0,)),
            pl.BlockSpec((DIM, 4 * DIM), lambda b: (0, 0)),
            pl.BlockSpec((4 * DIM,), lambda b: (0,)),
            pl.BlockSpec((4 * DIM, DIM), lambda b: (0, 0)),
            pl.BlockSpec((DIM,), lambda b: (0,)),
        ],
        out_specs=pl.BlockSpec((1, DIM, H * W), lambda b: (b, 0, 0)),
        out_shape=jax.ShapeDtypeStruct((n, DIM, H * W), jnp.float32),
    )(x_flat, o_img, v_img, lw, lepe_b, gamma1, gamma2, ln2_g, ln2_b,
      mlp_w1, mlp_b1, mlp_w2, mlp_b2)

    return out.reshape(n, DIM, H, W)
